# baseline (speedup 1.0000x reference)
import jax
import jax.numpy as jnp
from jax.experimental import pallas as pl
from jax.experimental.pallas import tpu as pltpu

_NE = 1024
_D = 64
_BLK = 512
_COMMIT = 0.25


def _vq_tc_body(x_ref, w_ref, idx_ref, qst_ref, acc_ref):
    i = pl.program_id(0)
    x = x_ref[...]
    w = w_ref[...]
    x2 = jnp.sum(x * x, axis=1, keepdims=True)
    w2 = jnp.sum(w * w, axis=1)
    m = jax.lax.dot_general(x, w, (((1,), (1,)), ((), ())))
    d = (x2 + w2[None, :]) - 2.0 * m
    dmin = jnp.min(d, axis=1, keepdims=True)
    col = jax.lax.broadcasted_iota(jnp.int32, d.shape, 1)
    idx = jnp.min(jnp.where(d == dmin, col, _NE), axis=1)
    idx_ref[...] = idx[None, None, :]
    oh = (col == idx[:, None]).astype(jnp.float32)
    q = jax.lax.dot_general(oh, w, (((1,), (0,)), ((), ())),
                            precision=jax.lax.Precision.HIGHEST)
    qst_ref[...] = x + (q - x)

    @pl.when(i == 0)
    def _():
        acc_ref[...] = jnp.zeros_like(acc_ref)

    s = jnp.sum(dmin)
    acc_ref[...] += jnp.full((8, 128), s * (1.0 / 1024.0), jnp.float32)


def kernel(inputs, weight):
    x = inputs.reshape(-1, _D)
    n = x.shape[0]
    nb = n // _BLK
    idx3, qst, acc = pl.pallas_call(
        _vq_tc_body,
        grid=(nb,),
        in_specs=[pl.BlockSpec((_BLK, _D), lambda i: (i, 0)),
                  pl.BlockSpec((_NE, _D), lambda i: (0, 0))],
        out_specs=[pl.BlockSpec((1, 1, _BLK), lambda i: (i, 0, 0)),
                   pl.BlockSpec((_BLK, _D), lambda i: (i, 0)),
                   pl.BlockSpec((8, 128), lambda i: (0, 0))],
        out_shape=[jax.ShapeDtypeStruct((nb, 1, _BLK), jnp.int32),
                   jax.ShapeDtypeStruct((n, _D), jnp.float32),
                   jax.ShapeDtypeStruct((8, 128), jnp.float32)],
    )(x, weight)
    e = jnp.sum(acc) / (n * _D)
    loss = e + _COMMIT * e
    return loss, qst.reshape(inputs.shape), idx3.reshape(inputs.shape[:-1])


# speedup vs baseline: 1.4691x; 1.4691x over previous
import functools

import jax
import jax.numpy as jnp
from jax import lax
from jax.experimental import pallas as pl
from jax.experimental.pallas import tpu as pltpu
from jax.experimental.pallas import tpu_sc as plsc

_NE = 1024
_D = 64
_BLK = 512
_COMMIT = 0.25

_NW = 32
_CHUNK = 128


def _vq_tc_body(x_ref, w_ref, idx_ref, acc_ref):
    i = pl.program_id(0)
    x = x_ref[...]
    w = w_ref[...]
    x2 = jnp.sum(x * x, axis=1, keepdims=True)
    w2 = jnp.sum(w * w, axis=1)
    m = lax.dot_general(x, w, (((1,), (1,)), ((), ())))
    d = (x2 + w2[None, :]) - 2.0 * m
    dmin = jnp.min(d, axis=1, keepdims=True)
    col = lax.broadcasted_iota(jnp.int32, d.shape, 1)
    idx = jnp.min(jnp.where(d == dmin, col, _NE), axis=1)
    idx_ref[...] = idx[None, None, :]

    @pl.when(i == 0)
    def _():
        acc_ref[...] = jnp.zeros_like(acc_ref)

    s = jnp.sum(dmin)
    acc_ref[...] += jnp.full((8, 128), s * (1.0 / 1024.0), jnp.float32)


def _sc_gather_body(w_hbm, idx_hbm, out_hbm, idx_v, rows_v, sem):
    wid = lax.axis_index("s") * 2 + lax.axis_index("c")
    nchunk = idx_v.shape[0]
    nbuf = rows_v.shape[0]
    pltpu.sync_copy(idx_hbm.at[wid], idx_v)
    copies = [pltpu.async_copy(w_hbm.at[idx_v.at[j]], rows_v.at[j], sem)
              for j in range(nbuf)]
    for j in range(nchunk):
        copies[j % nbuf].wait()
        pltpu.sync_copy(rows_v.at[j % nbuf],
                        out_hbm.at[wid].at[pl.ds(j * _CHUNK, _CHUNK)])
        nxt = j + nbuf
        if nxt < nchunk:
            copies[nxt % nbuf] = pltpu.async_copy(
                w_hbm.at[idx_v.at[nxt]], rows_v.at[nxt % nbuf], sem)


def kernel(inputs, weight):
    x = inputs.reshape(-1, _D)
    n = x.shape[0]
    nb = n // _BLK
    idx3, acc = pl.pallas_call(
        _vq_tc_body,
        grid=(nb,),
        in_specs=[pl.BlockSpec((_BLK, _D), lambda i: (i, 0)),
                  pl.BlockSpec((_NE, _D), lambda i: (0, 0))],
        out_specs=[pl.BlockSpec((1, 1, _BLK), lambda i: (i, 0, 0)),
                   pl.BlockSpec((8, 128), lambda i: (0, 0))],
        out_shape=[jax.ShapeDtypeStruct((nb, 1, _BLK), jnp.int32),
                   jax.ShapeDtypeStruct((8, 128), jnp.float32)],
    )(x, weight)

    b_per_w = n // _NW
    nchunk = b_per_w // _CHUNK
    idx_sc = idx3.reshape(_NW, nchunk, _CHUNK)
    w_pad = jnp.pad(weight, ((0, 0), (0, 128 - _D)))
    mesh = plsc.VectorSubcoreMesh(core_axis_name="c", subcore_axis_name="s")
    q = pl.kernel(
        _sc_gather_body,
        out_type=jax.ShapeDtypeStruct((_NW, b_per_w, 128), jnp.float32),
        mesh=mesh,
        scratch_types=[
            pltpu.VMEM((nchunk, _CHUNK), jnp.int32),
            pltpu.VMEM((3, _CHUNK, 128), jnp.float32),
            pltpu.SemaphoreType.DMA,
        ],
    )(w_pad, idx_sc)

    e = jnp.sum(acc) / (n * _D)
    loss = e + _COMMIT * e
    q64 = q.reshape(n, 128)[:, :_D]
    return loss, q64.reshape(inputs.shape), idx3.reshape(inputs.shape[:-1])


# speedup vs baseline: 1.6012x; 1.0900x over previous
import functools

import jax
import jax.numpy as jnp
from jax import lax
from jax.experimental import pallas as pl
from jax.experimental.pallas import tpu as pltpu
from jax.experimental.pallas import tpu_sc as plsc

_NE = 1024
_D = 64
_BLK = 512
_COMMIT = 0.25

_NW = 32
_CHUNK = 128


def _vq_tc_body(x_ref, w_ref, idx_ref, acc_ref, w2_ref):
    i = pl.program_id(0)

    @pl.when(i == 0)
    def _():
        w0 = w_ref[...]
        w2_ref[...] = (0.5 * jnp.sum(w0 * w0, axis=1))[None, :]
        acc_ref[...] = jnp.zeros_like(acc_ref)

    x = x_ref[...]
    x2 = jnp.sum(x * x, axis=1, keepdims=True)
    m = lax.dot_general(x, w_ref[...], (((1,), (1,)), ((), ())))
    score = m - w2_ref[...]
    smax = jnp.max(score, axis=1, keepdims=True)
    col = lax.broadcasted_iota(jnp.int32, (1, _NE), 1)
    idx = jnp.min(jnp.where(score == smax, col, _NE), axis=1)
    idx_ref[...] = idx[None, None, :]

    s = jnp.sum(x2) - 2.0 * jnp.sum(smax)
    acc_ref[...] += jnp.full((8, 128), s * (1.0 / 1024.0), jnp.float32)


def _sc_gather_body(w_hbm, idx_hbm, out_hbm, idx_v, rows_v, sem):
    wid = lax.axis_index("s") * 2 + lax.axis_index("c")
    nchunk = idx_v.shape[0]
    nbuf = rows_v.shape[0]
    pltpu.sync_copy(idx_hbm.at[wid], idx_v)
    copies = [pltpu.async_copy(w_hbm.at[idx_v.at[j]], rows_v.at[j], sem)
              for j in range(nbuf)]
    for j in range(nchunk):
        copies[j % nbuf].wait()
        pltpu.sync_copy(rows_v.at[j % nbuf],
                        out_hbm.at[wid].at[pl.ds(j * _CHUNK, _CHUNK)])
        nxt = j + nbuf
        if nxt < nchunk:
            copies[nxt % nbuf] = pltpu.async_copy(
                w_hbm.at[idx_v.at[nxt]], rows_v.at[nxt % nbuf], sem)


def kernel(inputs, weight):
    x = inputs.reshape(-1, _D)
    n = x.shape[0]
    nb = n // _BLK
    idx3, acc = pl.pallas_call(
        _vq_tc_body,
        grid=(nb,),
        in_specs=[pl.BlockSpec((_BLK, _D), lambda i: (i, 0)),
                  pl.BlockSpec((_NE, _D), lambda i: (0, 0))],
        out_specs=[pl.BlockSpec((1, 1, _BLK), lambda i: (i, 0, 0)),
                   pl.BlockSpec((8, 128), lambda i: (0, 0))],
        out_shape=[jax.ShapeDtypeStruct((nb, 1, _BLK), jnp.int32),
                   jax.ShapeDtypeStruct((8, 128), jnp.float32)],
        scratch_shapes=[pltpu.VMEM((1, _NE), jnp.float32)],
    )(x, weight)

    b_per_w = n // _NW
    nchunk = b_per_w // _CHUNK
    idx_sc = idx3.reshape(_NW, nchunk, _CHUNK)
    w_pad = jnp.pad(weight, ((0, 0), (0, 128 - _D)))
    mesh = plsc.VectorSubcoreMesh(core_axis_name="c", subcore_axis_name="s")
    q = pl.kernel(
        _sc_gather_body,
        out_type=jax.ShapeDtypeStruct((_NW, b_per_w, 128), jnp.float32),
        mesh=mesh,
        scratch_types=[
            pltpu.VMEM((nchunk, _CHUNK), jnp.int32),
            pltpu.VMEM((3, _CHUNK, 128), jnp.float32),
            pltpu.SemaphoreType.DMA,
        ],
    )(w_pad, idx_sc)

    e = jnp.sum(acc) / (n * _D)
    loss = e + _COMMIT * e
    q64 = q.reshape(n, 128)[:, :_D]
    return loss, q64.reshape(inputs.shape), idx3.reshape(inputs.shape[:-1])
